# baseline (speedup 1.0000x reference)
import functools

import jax
import jax.numpy as jnp
from jax import lax
from jax.experimental import pallas as pl
from jax.experimental.pallas import tpu as pltpu
from jax.experimental.pallas import tpu_sc as plsc

_VOCAB = 100000
_SYN_VOCAB = 1024
_D = 64
_PAD_D = 16
_LANG_D = 8
_B, _L = 8, 64
_TOK = _B * _L
_SCALE = 8.0

_NC, _NS = 2, 16
_NW = _NC * _NS
_TPW = _TOK // _NW

_TILE = 8192
_GRID = (_VOCAB + _TILE - 1) // _TILE


def _sc_onehot_body(idx_hbm, oe_hbm, idx_v, e_v):
    wid = lax.axis_index("s") * _NC + lax.axis_index("c")
    base = wid * _TPW
    pltpu.sync_copy(idx_hbm.at[pl.ds(base, _TPW)], idx_v)
    lane = lax.iota(jnp.int32, 16)
    zeros16 = jnp.zeros((16,), jnp.float32)
    for t in range(_TPW):
        for j in range(_SYN_VOCAB // 16):
            e_v[t, pl.ds(j * 16, 16)] = zeros16
    plsc.store_scatter(e_v, [lane, idx_v[...] & (_SYN_VOCAB - 1)],
                       jnp.ones((16,), jnp.float32))
    pltpu.sync_copy(e_v, oe_hbm.at[pl.ds(base, _TPW)])


@functools.lru_cache(maxsize=1)
def _sc_onehot_kernel():
    return pl.kernel(
        _sc_onehot_body,
        out_type=jax.ShapeDtypeStruct((_TOK, _SYN_VOCAB), jnp.float32),
        mesh=plsc.VectorSubcoreMesh(core_axis_name="c", subcore_axis_name="s"),
        scratch_types=[
            pltpu.VMEM((_TPW,), jnp.int32),
            pltpu.VMEM((_TPW, _SYN_VOCAB), jnp.float32),
        ],
        compiler_params=pltpu.CompilerParams(needs_layout_passes=False),
    )


def _sc_onehot(idx):
    return _sc_onehot_kernel()(idx)


def _three_stage(x):
    x = x * _SCALE
    steep = 3.0
    pos = jax.nn.sigmoid(steep * (x - 4.0))
    neg = jax.nn.sigmoid(steep * (-x - 4.0))
    return pos - neg


def _tc_body(ids_sref, t1_any, t2_any, wse_ref, bse_ref, wss_ref, bss_ref,
             wrev_ref, brev_ref,
             big_ref, sym_ref, syn_ref, ssum_ref,
             e1_v, e2_v, sp_ref, syn_sc_ref, sem1, sem2):
    i = pl.program_id(0)

    @pl.when(i == 0)
    def _critical():
        def issue(j, carry):
            row = ids_sref[j]
            pltpu.make_async_copy(
                t1_any.at[pl.ds(row, 1)], e1_v.at[pl.ds(j, 1)], sem1).start()
            pltpu.make_async_copy(
                t2_any.at[pl.ds(row, 1)], e2_v.at[pl.ds(j, 1)], sem2).start()
            return carry
        lax.fori_loop(0, _TOK, issue, 0)
        pltpu.make_async_copy(t1_any.at[pl.ds(0, _TOK)], e1_v, sem1).wait()
        pltpu.make_async_copy(t2_any.at[pl.ds(0, _TOK)], e2_v, sem2).wait()
        syn = _three_stage(e2_v[...] * (1.0 / _SCALE))
        sfs = jnp.dot(syn, wse_ref[...],
                      preferred_element_type=jnp.float32) + bse_ref[...]
        sp_ref[...] = e1_v[...] + sfs
        syn_sc_ref[...] = syn

    @pl.when(i == 1)
    def _small():
        syn = syn_sc_ref[...]
        sym_ref[...] = sp_ref[...]
        syn_ref[...] = syn
        ssum_ref[...] = jnp.dot(syn, wss_ref[...],
                                preferred_element_type=jnp.float32) + bss_ref[...]

    wr = wrev_ref[...]
    bias = 0.1 * jnp.sum(wr[_D:, :], axis=0, keepdims=True) + brev_ref[...]
    big_ref[...] = (
        jnp.dot(sp_ref[...].astype(jnp.bfloat16),
                wr[:_D, :].astype(jnp.bfloat16),
                preferred_element_type=jnp.float32)
        + bias)


def _tc_dense(ids_flat, t1, t2, wse, bse, wss, bss, wrev, brev):
    grid_spec = pltpu.PrefetchScalarGridSpec(
        num_scalar_prefetch=1,
        grid=(_GRID,),
        in_specs=[
            pl.BlockSpec(memory_space=pl.ANY),
            pl.BlockSpec(memory_space=pl.ANY),
            pl.BlockSpec((_D, _D), lambda i, ids: (0, 0)),
            pl.BlockSpec((1, _D), lambda i, ids: (0, 0)),
            pl.BlockSpec((_D, _SYN_VOCAB), lambda i, ids: (0, 0)),
            pl.BlockSpec((1, _SYN_VOCAB), lambda i, ids: (0, 0)),
            pl.BlockSpec((_D + _PAD_D, _TILE), lambda i, ids: (0, i)),
            pl.BlockSpec((1, _TILE), lambda i, ids: (0, i)),
        ],
        out_specs=[
            pl.BlockSpec((_TOK, _TILE), lambda i, ids: (0, i)),
            pl.BlockSpec((_TOK, _D), lambda i, ids: (0, 0)),
            pl.BlockSpec((_TOK, _D), lambda i, ids: (0, 0)),
            pl.BlockSpec((_TOK, _SYN_VOCAB), lambda i, ids: (0, 0)),
        ],
        scratch_shapes=[
            pltpu.VMEM((_TOK, _D), jnp.float32),
            pltpu.VMEM((_TOK, _D), jnp.float32),
            pltpu.VMEM((_TOK, _D), jnp.float32),
            pltpu.VMEM((_TOK, _D), jnp.float32),
            pltpu.SemaphoreType.DMA,
            pltpu.SemaphoreType.DMA,
        ],
    )
    return pl.pallas_call(
        _tc_body,
        grid_spec=grid_spec,
        out_shape=[
            jax.ShapeDtypeStruct((_TOK, _VOCAB), jnp.float32),
            jax.ShapeDtypeStruct((_TOK, _D), jnp.float32),
            jax.ShapeDtypeStruct((_TOK, _D), jnp.float32),
            jax.ShapeDtypeStruct((_TOK, _SYN_VOCAB), jnp.float32),
        ],
        compiler_params=pltpu.CompilerParams(
            vmem_limit_bytes=100 * 1024 * 1024),
    )(ids_flat, t1, t2, wse, bse, wss, bss, wrev, brev)


def kernel(ids, table_v1, table_v2, W_syn_emb, b_syn_emb, W_syn_sum,
           b_syn_sum, W_rev, b_rev):
    ids_flat = ids.reshape(_TOK).astype(jnp.int32)
    exp = _sc_onehot(ids_flat)
    big, symbol, syn, ssum = _tc_dense(
        ids_flat, table_v1, table_v2,
        W_syn_emb, b_syn_emb.reshape(1, _D),
        W_syn_sum, b_syn_sum.reshape(1, _SYN_VOCAB),
        W_rev, b_rev.reshape(1, _VOCAB))
    pad_lang = jnp.full((_B, _L, _PAD_D + _LANG_D), 0.1, dtype=jnp.float32)
    full = jnp.concatenate(
        [symbol.reshape(_B, _L, _D), pad_lang, syn.reshape(_B, _L, _D)],
        axis=2)
    return (full,
            big.reshape(_B, _L, _VOCAB),
            ssum.reshape(_B, _L, _SYN_VOCAB),
            exp.reshape(_B, _L, _SYN_VOCAB))
